# baseline (speedup 1.0000x reference)
import numpy as np
import jax
import jax.numpy as jnp
from jax import lax
from jax.experimental import pallas as pl
from jax.experimental.pallas import tpu as pltpu

BATCH = 8
SEQ = 12
FEAT = 64
NODES = 128
KS = 3
DILS = (1, 2)
T0 = 4
NT = SEQ - T0
RR = BATCH * NT
BH = BATCH // 2

_HI = lax.Precision.HIGHEST
_DF = lax.Precision.DEFAULT

_W0, _W1, _CW0, _CW1, _BIAS, _PSEL_OFF = 0, 64, 128, 320, 512, 520
_PACK_ROWS = 528

_PSEL = np.zeros((BATCH, RR), dtype=np.float32)
for _b in range(BATCH):
    _PSEL[_b, _b * NT + (NT - 1)] = 0.5


def _fused_kernel(nea_ref, neb_ref, pack_ref, out_ref, mu_ref):
    i = pl.program_id(0)
    mua = jnp.mean(nea_ref[...], axis=-1)
    mub = jnp.mean(neb_ref[...], axis=-1)
    half_rows = BH * NT
    mu_ref[pl.ds(i * half_rows, half_rows), :] = jnp.reshape(
        jnp.concatenate([mua, mub], axis=1), (half_rows, FEAT))

    @pl.when(i == 1)
    def _finish():
        tidx = lax.broadcasted_iota(jnp.int32, (RR, 1), 0) % NT
        mu0 = mu_ref[...]

        def shift(x, s):
            if s == 0:
                return x
            return jnp.where(tidx >= s, pltpu.roll(x, s, 0), 0.0)

        def causal_conv(x, woff, boff, cwoff, cboff, d):
            w = pack_ref[woff:woff + FEAT, :]
            brow = pack_ref[boff:boff + 1, :]
            acc = jnp.zeros((RR, FEAT), jnp.float32)
            bias = pack_ref[cboff:cboff + 1, :]
            for k in range(KS):
                s = (KS - 1 - k) * d
                mk = pack_ref[cwoff + FEAT * k:cwoff + FEAT * (k + 1), :]
                wk = jnp.dot(w, mk, precision=_HI)
                bk = jnp.dot(brow, mk, precision=_HI)
                acc = acc + jnp.dot(shift(x, s), wk, precision=_DF)
                bias = bias + jnp.where(tidx >= s, bk, 0.0)
            return jax.nn.relu(acc + bias)

        c0 = causal_conv(mu0, _W0, _BIAS + 0, _CW0, _BIAS + 2, DILS[0])
        c1 = causal_conv(mu0 + c0, _W1, _BIAS + 1, _CW1, _BIAS + 3, DILS[1])

        psel = pack_ref[_PSEL_OFF:_PSEL_OFF + BATCH, :]
        y = jnp.dot(psel, c0 + c1, precision=_DF)
        out_ref[...] = jnp.broadcast_to(y[:, None, :], (BATCH, NODES, FEAT))


def kernel(node_embeddings, B, static_MTE_matrix, batch_index, use_MTE,
           is_training, learnable_adj, W_gcn0, b_gcn0, W_gcn1, b_gcn1,
           conv_w0, conv_b0, conv_w1, conv_b1, Wa, ba, v):
    pack = jnp.concatenate([
        W_gcn0,
        W_gcn1,
        jnp.transpose(conv_w0[:, :, 0, :], (2, 1, 0)).reshape(KS * FEAT, FEAT),
        jnp.transpose(conv_w1[:, :, 0, :], (2, 1, 0)).reshape(KS * FEAT, FEAT),
        b_gcn0.reshape(1, FEAT),
        b_gcn1.reshape(1, FEAT),
        conv_b0.reshape(1, FEAT),
        conv_b1.reshape(1, FEAT),
        jnp.zeros((4, FEAT), jnp.float32),
        jnp.asarray(_PSEL),
    ], axis=0)

    half_t = NT // 2
    out = pl.pallas_call(
        _fused_kernel,
        grid=(2,),
        in_specs=[
            pl.BlockSpec((BH, half_t, FEAT, NODES), lambda i: (i, 1, 0, 0)),
            pl.BlockSpec((BH, half_t, FEAT, NODES), lambda i: (i, 2, 0, 0)),
            pl.BlockSpec((_PACK_ROWS, FEAT), lambda i: (0, 0)),
        ],
        out_specs=pl.BlockSpec((BATCH, NODES, FEAT), lambda i: (0, 0, 0)),
        out_shape=jax.ShapeDtypeStruct((BATCH, NODES, FEAT), jnp.float32),
        scratch_shapes=[pltpu.VMEM((RR, FEAT), jnp.float32)],
    )(node_embeddings, node_embeddings, pack)
    return out
